# baseline (speedup 1.0000x reference)
import functools

import jax
import jax.numpy as jnp
from jax import lax
from jax.experimental import pallas as pl
from jax.experimental.pallas import tpu as pltpu
from jax.experimental.pallas import tpu_sc as plsc

N_USERS = 100000
N_ITEMS = 100000
D = 64
B = 4096
NEG = 200
K_NBR = 10
W1 = 1e-06
W2 = 1.0
W3 = 1e-06
W4 = 1.0
NEG_W = 200.0
GAMMA = 1e-04
LAMBDA_W = 2.75

NC = 2
NS = 16
NW = NC * NS
BPT = B // NW
NEG_PAD = 208
NGRP = NEG_PAD // 16
SPLIT = 104


def _sc_body(user_hbm, pos_hbm, pos2_hbm, neg_hbm, user_emb, item_emb,
             beta_u16_hbm, beta_i16_hbm, il_neighbor, neg_logit_hbm,
             neg_bi_hbm, small_hbm, nbr_hbm, bu_hbm, bip_hbm,
             user_idx, pos_idx, pos2_idx, neg_idx, nbr_idx, u_rows, bu_loc,
             bip_loc, rows_a, rows_b, rows2_a, rows2_b, beta16_a, beta16_b,
             hi_a, hi_b, lrow_a, lrow_b, brow_a, brow_b, srow_a, srow_b,
             sem_pro, sem_a, sem_b):
    wid = lax.axis_index("s") * NC + lax.axis_index("c")
    base = pl.multiple_of(wid * BPT, BPT)
    iota = lax.iota(jnp.int32, 16)

    pltpu.sync_copy(user_hbm.at[pl.ds(base, BPT)], user_idx)
    pltpu.sync_copy(pos_hbm.at[pl.ds(base, BPT)], pos_idx)
    pltpu.sync_copy(pos2_hbm.at[pl.ds(base, BPT)], pos2_idx)
    pltpu.sync_copy(neg_hbm.at[pl.ds(base, BPT)], neg_idx)
    c1 = pltpu.async_copy(user_emb.at[user_idx], u_rows, sem_pro)
    c2 = pltpu.async_copy(il_neighbor.at[pos_idx], nbr_idx, sem_pro)
    c1.wait(); c2.wait()

    for tbl_hbm, idx_ref, dst in ((beta_u16_hbm, user_idx, bu_loc),
                                  (beta_i16_hbm, pos_idx, bip_loc)):
        for c in range(BPT // 16):
            hi_a[pl.ds(c * 16, 16)] = \
                lax.shift_right_logical(idx_ref[pl.ds(c * 16, 16)], 4)
        pltpu.async_copy(tbl_hbm.at[hi_a.at[pl.ds(0, 128)]],
                         beta16_a.at[pl.ds(0, 128)], sem_pro).wait()
        for c in range(BPT // 16):
            lo = idx_ref[pl.ds(c * 16, 16)] & 15
            dst[pl.ds(c * 16, 16)] = plsc.load_gather(
                beta16_a, [c * 16 + iota, lo])

    row_idx = [jnp.minimum(g * 16 + iota, NEG - 1) for g in range(NGRP)]
    row_idx_s = jnp.minimum(iota, K_NBR)
    grp_rows = [g * 16 + iota for g in range(NGRP)]

    def issue_gathers(i, rows_s, rows2_s, beta16_s, hi_s, sem):
        pltpu.async_copy(
            item_emb.at[neg_idx.at[i, pl.ds(0, SPLIT)]],
            rows_s.at[pl.ds(0, SPLIT)], sem)
        pltpu.async_copy(
            item_emb.at[neg_idx.at[i, pl.ds(SPLIT, NEG - SPLIT)]],
            rows_s.at[pl.ds(SPLIT, NEG - SPLIT)], sem)
        pltpu.async_copy(
            item_emb.at[pos2_idx.at[i]],
            rows2_s.at[pl.ds(0, 1)], sem)
        pltpu.async_copy(
            item_emb.at[nbr_idx.at[i]],
            rows2_s.at[pl.ds(1, 16)], sem)
        for g in range(NGRP):
            hi_s[pl.ds(g * 16, 16)] = \
                lax.shift_right_logical(neg_idx[i, pl.ds(g * 16, 16)], 4)
        pltpu.async_copy(
            beta_i16_hbm.at[hi_s.at[pl.ds(0, SPLIT)]],
            beta16_s.at[pl.ds(0, SPLIT)], sem)
        pltpu.async_copy(
            beta_i16_hbm.at[hi_s.at[pl.ds(SPLIT, NEG_PAD - SPLIT)]],
            beta16_s.at[pl.ds(SPLIT, NEG_PAD - SPLIT)], sem)

    def issue_stores(i, lrow, brow, srow, sem):
        pltpu.async_copy(lrow, neg_logit_hbm.at[pl.ds(base + i, 1)], sem)
        pltpu.async_copy(brow, neg_bi_hbm.at[pl.ds(base + i, 1)], sem)
        pltpu.async_copy(srow, small_hbm.at[pl.ds(base + i, 1)], sem)

    def drain(rows_s, rows2_s, beta16_s, hi_s, lrow, brow, srow, sem):
        pltpu.make_async_copy(
            item_emb.at[neg_idx.at[0, pl.ds(0, SPLIT)]],
            rows_s.at[pl.ds(0, SPLIT)], sem).wait()
        pltpu.make_async_copy(
            item_emb.at[neg_idx.at[0, pl.ds(SPLIT, NEG - SPLIT)]],
            rows_s.at[pl.ds(SPLIT, NEG - SPLIT)], sem).wait()
        pltpu.make_async_copy(
            item_emb.at[pos2_idx.at[0]],
            rows2_s.at[pl.ds(0, 1)], sem).wait()
        pltpu.make_async_copy(
            item_emb.at[nbr_idx.at[0]],
            rows2_s.at[pl.ds(1, 16)], sem).wait()
        pltpu.make_async_copy(
            beta_i16_hbm.at[hi_s.at[pl.ds(0, SPLIT)]],
            beta16_s.at[pl.ds(0, SPLIT)], sem).wait()
        pltpu.make_async_copy(
            beta_i16_hbm.at[hi_s.at[pl.ds(SPLIT, NEG_PAD - SPLIT)]],
            beta16_s.at[pl.ds(SPLIT, NEG_PAD - SPLIT)], sem).wait()
        drain_stores(lrow, brow, srow, sem)

    def drain_stores(lrow, brow, srow, sem):
        pltpu.make_async_copy(
            lrow, neg_logit_hbm.at[pl.ds(base, 1)], sem).wait()
        pltpu.make_async_copy(
            brow, neg_bi_hbm.at[pl.ds(base, 1)], sem).wait()
        pltpu.make_async_copy(
            srow, small_hbm.at[pl.ds(base, 1)], sem).wait()

    def compute(i, rows_s, rows2_s, beta16_s, lrow, brow, srow):
        zero = jnp.zeros((16,), jnp.float32)
        accs = [zero] * NGRP
        acc_s = zero
        u_chunks = [u_rows[i, pl.ds(c * 16, 16)] for c in range(D // 16)]
        for d in range(D):
            u_val = u_chunks[d // 16][d % 16]
            dv = jnp.full((16,), d, jnp.int32)
            uv = jnp.full((16,), 1.0, jnp.float32) * u_val
            for g in range(NGRP):
                vals = plsc.load_gather(rows_s, [row_idx[g], dv])
                accs[g] = accs[g] + vals * uv
            vals2 = plsc.load_gather(rows2_s, [row_idx_s, dv])
            acc_s = acc_s + vals2 * uv
        for g in range(NGRP):
            lrow[0, pl.ds(g * 16, 16)] = accs[g]
            lo = neg_idx[i, pl.ds(g * 16, 16)] & 15
            brow[0, pl.ds(g * 16, 16)] = plsc.load_gather(
                beta16_s, [grp_rows[g], lo])
        srow[0, :] = acc_s

    issue_gathers(0, rows_a, rows2_a, beta16_a, hi_a, sem_a)
    issue_stores(0, lrow_a, brow_a, srow_a, sem_a)
    issue_stores(1, lrow_b, brow_b, srow_b, sem_b)

    def step2(j, _):
        i0 = 2 * j
        issue_gathers(i0 + 1, rows_b, rows2_b, beta16_b, hi_b, sem_b)
        drain(rows_a, rows2_a, beta16_a, hi_a, lrow_a, brow_a, srow_a, sem_a)
        compute(i0, rows_a, rows2_a, beta16_a, lrow_a, brow_a, srow_a)
        issue_stores(i0, lrow_a, brow_a, srow_a, sem_a)
        issue_gathers(jnp.minimum(i0 + 2, BPT - 1), rows_a, rows2_a,
                      beta16_a, hi_a, sem_a)
        drain(rows_b, rows2_b, beta16_b, hi_b, lrow_b, brow_b, srow_b, sem_b)
        compute(i0 + 1, rows_b, rows2_b, beta16_b, lrow_b, brow_b, srow_b)
        issue_stores(i0 + 1, lrow_b, brow_b, srow_b, sem_b)
        return _

    lax.fori_loop(0, BPT // 2, step2, None)
    drain(rows_a, rows2_a, beta16_a, hi_a, lrow_a, brow_a, srow_a, sem_a)
    drain_stores(lrow_b, brow_b, srow_b, sem_b)

    pltpu.sync_copy(nbr_idx, nbr_hbm.at[pl.ds(base, BPT)])
    pltpu.sync_copy(bu_loc, bu_hbm.at[pl.ds(base, BPT)])
    pltpu.sync_copy(bip_loc, bip_hbm.at[pl.ds(base, BPT)])


@functools.cache
def _get_sc_call():
  return pl.kernel(
    _sc_body,
    out_type=(
        jax.ShapeDtypeStruct((B, NEG_PAD), jnp.float32),
        jax.ShapeDtypeStruct((B, NEG_PAD), jnp.float32),
        jax.ShapeDtypeStruct((B, 16), jnp.float32),
        jax.ShapeDtypeStruct((B, 16), jnp.int32),
        jax.ShapeDtypeStruct((B,), jnp.float32),
        jax.ShapeDtypeStruct((B,), jnp.float32),
    ),
    mesh=plsc.VectorSubcoreMesh(core_axis_name="c", subcore_axis_name="s",
                                num_cores=NC, num_subcores=NS),
    scratch_types=[
        pltpu.VMEM((BPT,), jnp.int32),
        pltpu.VMEM((BPT,), jnp.int32),
        pltpu.VMEM((BPT, 1), jnp.int32),
        pltpu.VMEM((BPT, NEG_PAD), jnp.int32),
        pltpu.VMEM((BPT, 16), jnp.int32),
        pltpu.VMEM((BPT, D), jnp.float32),
        pltpu.VMEM((BPT,), jnp.float32),
        pltpu.VMEM((BPT,), jnp.float32),
        pltpu.VMEM((NEG, D), jnp.float32),
        pltpu.VMEM((NEG, D), jnp.float32),
        pltpu.VMEM((17, D), jnp.float32),
        pltpu.VMEM((17, D), jnp.float32),
        pltpu.VMEM((NEG_PAD, 16), jnp.float32),
        pltpu.VMEM((NEG_PAD, 16), jnp.float32),
        pltpu.VMEM((NEG_PAD,), jnp.int32),
        pltpu.VMEM((NEG_PAD,), jnp.int32),
        pltpu.VMEM((1, NEG_PAD), jnp.float32),
        pltpu.VMEM((1, NEG_PAD), jnp.float32),
        pltpu.VMEM((1, NEG_PAD), jnp.float32),
        pltpu.VMEM((1, NEG_PAD), jnp.float32),
        pltpu.VMEM((1, 16), jnp.float32),
        pltpu.VMEM((1, 16), jnp.float32),
        pltpu.SemaphoreType.DMA,
        pltpu.SemaphoreType.DMA,
        pltpu.SemaphoreType.DMA,
    ],
    compiler_params=pltpu.CompilerParams(use_tc_tiling_on_sc=False,
                                         needs_layout_passes=False),
  )


TBL_CHUNK = 5000
TBL_GRID = N_ITEMS // TBL_CHUNK


def _softplus(x):
    return jnp.maximum(x, 0.0) + jnp.log1p(jnp.exp(-jnp.abs(x)))


def _tc_body(ue_ref, ie_ref, nl_ref, nb_ref, sm_ref, nbr_ref, bu_ref,
             bip_ref, out_ref):
    i = pl.program_id(0)

    @pl.when(i == 0)
    def _():
        bu = bu_ref[...]
        neg_w = W3 + W4 * bu * nb_ref[:, :NEG]
        t_neg = jnp.sum(neg_w * _softplus(nl_ref[:, :NEG])) * (NEG_W / NEG)
        pos_w = W1 + W2 * bu * bip_ref[...]
        t_pos = jnp.sum(pos_w * _softplus(-sm_ref[:, 0:1]))
        nbr_f = nbr_ref[:, :K_NBR].astype(jnp.float32)
        t_il = LAMBDA_W * jnp.sum(nbr_f * _softplus(-sm_ref[:, 1:1 + K_NBR]))
        out_ref[0, 0] = t_pos + t_neg + t_il

    norm_part = jnp.sum(ue_ref[...] * ue_ref[...]) + \
        jnp.sum(ie_ref[...] * ie_ref[...])
    out_ref[0, 0] = out_ref[0, 0] + norm_part * (GAMMA * 0.5)


_tc_call = pl.pallas_call(
    _tc_body,
    out_shape=jax.ShapeDtypeStruct((1, 1), jnp.float32),
    grid=(TBL_GRID,),
    in_specs=[
        pl.BlockSpec((TBL_CHUNK, D), lambda i: (i, 0)),
        pl.BlockSpec((TBL_CHUNK, D), lambda i: (i, 0)),
        pl.BlockSpec((B, NEG_PAD), lambda i: (0, 0)),
        pl.BlockSpec((B, NEG_PAD), lambda i: (0, 0)),
        pl.BlockSpec((B, 16), lambda i: (0, 0)),
        pl.BlockSpec((B, 16), lambda i: (0, 0)),
        pl.BlockSpec((B, 1), lambda i: (0, 0)),
        pl.BlockSpec((B, 1), lambda i: (0, 0)),
    ],
    out_specs=pl.BlockSpec(memory_space=pltpu.SMEM),
)


def kernel(user, pos, neg, user_emb, item_emb, beta_uD, beta_iD, il_neighbor):
    neg_pad = jnp.pad(neg, ((0, 0), (0, NEG_PAD - NEG)))
    il_pad = jnp.pad(il_neighbor, ((0, 0), (0, 16 - K_NBR)))
    neg_logit, neg_bi, small, nbr, bu, bip = _get_sc_call()(
        user, pos, pos.reshape(B, 1), neg_pad, user_emb, item_emb,
        beta_uD.reshape(N_USERS // 16, 16), beta_iD.reshape(N_ITEMS // 16, 16),
        il_pad)
    out = _tc_call(user_emb, item_emb, neg_logit, neg_bi, small, nbr,
                   bu.reshape(B, 1), bip.reshape(B, 1))
    return out[0, 0]


# speedup vs baseline: 1.0821x; 1.0821x over previous
import functools

import jax
import jax.numpy as jnp
from jax import lax
from jax.experimental import pallas as pl
from jax.experimental.pallas import tpu as pltpu
from jax.experimental.pallas import tpu_sc as plsc

N_USERS = 100000
N_ITEMS = 100000
D = 64
B = 4096
NEG = 200
K_NBR = 10
W1 = 1e-06
W2 = 1.0
W3 = 1e-06
W4 = 1.0
NEG_W = 200.0
GAMMA = 1e-04
LAMBDA_W = 2.75

NC = 2
NS = 16
NW = NC * NS
BPT = B // NW
NEG_PAD = 208
NGRP = NEG_PAD // 16
SPLIT = 104


def _sc_body(user_hbm, pos_hbm, pos2_hbm, neg_hbm, user_emb, item_emb,
             beta_u16_hbm, beta_i16_hbm, il_neighbor, neg_logit_hbm,
             neg_bi_hbm, small_hbm, nbr_hbm, bu_hbm, bip_hbm,
             user_idx, pos_idx, pos2_idx, neg_idx, nbr_idx, u_rows, bu_loc,
             bip_loc, rows, rows2, bneg_all, out_logit, out_small, hi_buf,
             beta16, ubeta16, sem_pro, sem_in):
    wid = lax.axis_index("s") * NC + lax.axis_index("c")
    base = pl.multiple_of(wid * BPT, BPT)
    iota = lax.iota(jnp.int32, 16)

    pltpu.sync_copy(user_hbm.at[pl.ds(base, BPT)], user_idx)
    pltpu.sync_copy(pos_hbm.at[pl.ds(base, BPT)], pos_idx)
    pltpu.sync_copy(pos2_hbm.at[pl.ds(base, BPT)], pos2_idx)
    pltpu.sync_copy(neg_hbm.at[pl.ds(base, BPT)], neg_idx)
    c1 = pltpu.async_copy(user_emb.at[user_idx], u_rows, sem_pro)
    c2 = pltpu.async_copy(il_neighbor.at[pos_idx], nbr_idx, sem_pro)
    c1.wait(); c2.wait()

    for tbl_hbm, idx_ref, dst in ((beta_u16_hbm, user_idx, bu_loc),
                                  (beta_i16_hbm, pos_idx, bip_loc)):
        for c in range(BPT // 16):
            hi_buf[pl.ds(c * 16, 16)] = \
                lax.shift_right_logical(idx_ref[pl.ds(c * 16, 16)], 4)
        pltpu.async_copy(tbl_hbm.at[hi_buf.at[pl.ds(0, 128)]], ubeta16,
                         sem_pro).wait()
        for c in range(BPT // 16):
            lo = idx_ref[pl.ds(c * 16, 16)] & 15
            dst[pl.ds(c * 16, 16)] = plsc.load_gather(
                ubeta16, [c * 16 + iota, lo])

    row_idx = [jnp.minimum(g * 16 + iota, NEG - 1) for g in range(NGRP)]
    row_idx_s = jnp.minimum(iota, K_NBR)
    grp_rows = [g * 16 + iota for g in range(NGRP)]

    def step(i, _):
        d1 = pltpu.async_copy(
            item_emb.at[neg_idx.at[i, pl.ds(0, SPLIT)]],
            rows.at[pl.ds(0, SPLIT)], sem_in)
        d2 = pltpu.async_copy(
            item_emb.at[neg_idx.at[i, pl.ds(SPLIT, NEG - SPLIT)]],
            rows.at[pl.ds(SPLIT, NEG - SPLIT)], sem_in)
        d3 = pltpu.async_copy(
            item_emb.at[pos2_idx.at[i]],
            rows2.at[pl.ds(0, 1)], sem_in)
        d4 = pltpu.async_copy(
            item_emb.at[nbr_idx.at[i]],
            rows2.at[pl.ds(1, 16)], sem_in)
        for g in range(NGRP):
            hi_buf[pl.ds(g * 16, 16)] = \
                lax.shift_right_logical(neg_idx[i, pl.ds(g * 16, 16)], 4)
        d5 = pltpu.async_copy(
            beta_i16_hbm.at[hi_buf.at[pl.ds(0, SPLIT)]],
            beta16.at[pl.ds(0, SPLIT)], sem_in)
        d6 = pltpu.async_copy(
            beta_i16_hbm.at[hi_buf.at[pl.ds(SPLIT, NEG_PAD - SPLIT)]],
            beta16.at[pl.ds(SPLIT, NEG_PAD - SPLIT)], sem_in)
        d1.wait(); d2.wait(); d3.wait(); d4.wait(); d5.wait(); d6.wait()

        zero = jnp.zeros((16,), jnp.float32)
        accs = [zero] * NGRP
        acc_s = zero
        u_chunks = [u_rows[i, pl.ds(c * 16, 16)] for c in range(D // 16)]
        for d in range(D):
            u_val = u_chunks[d // 16][d % 16]
            dv = jnp.full((16,), d, jnp.int32)
            uv = jnp.full((16,), 1.0, jnp.float32) * u_val
            for g in range(NGRP):
                vals = plsc.load_gather(rows, [row_idx[g], dv])
                accs[g] = accs[g] + vals * uv
            vals2 = plsc.load_gather(rows2, [row_idx_s, dv])
            acc_s = acc_s + vals2 * uv
        for g in range(NGRP):
            out_logit[i, pl.ds(g * 16, 16)] = accs[g]
            lo = neg_idx[i, pl.ds(g * 16, 16)] & 15
            bneg_all[i, pl.ds(g * 16, 16)] = plsc.load_gather(
                beta16, [grp_rows[g], lo])
        out_small[i, :] = acc_s
        return _

    lax.fori_loop(0, BPT, step, None)

    pltpu.sync_copy(out_logit, neg_logit_hbm.at[pl.ds(base, BPT)])
    pltpu.sync_copy(bneg_all, neg_bi_hbm.at[pl.ds(base, BPT)])
    pltpu.sync_copy(out_small, small_hbm.at[pl.ds(base, BPT)])
    pltpu.sync_copy(nbr_idx, nbr_hbm.at[pl.ds(base, BPT)])
    pltpu.sync_copy(bu_loc, bu_hbm.at[pl.ds(base, BPT)])
    pltpu.sync_copy(bip_loc, bip_hbm.at[pl.ds(base, BPT)])


@functools.cache
def _get_sc_call():
  return pl.kernel(
    _sc_body,
    out_type=(
        jax.ShapeDtypeStruct((B, NEG_PAD), jnp.float32),
        jax.ShapeDtypeStruct((B, NEG_PAD), jnp.float32),
        jax.ShapeDtypeStruct((B, 16), jnp.float32),
        jax.ShapeDtypeStruct((B, 16), jnp.int32),
        jax.ShapeDtypeStruct((B,), jnp.float32),
        jax.ShapeDtypeStruct((B,), jnp.float32),
    ),
    mesh=plsc.VectorSubcoreMesh(core_axis_name="c", subcore_axis_name="s",
                                num_cores=NC, num_subcores=NS),
    scratch_types=[
        pltpu.VMEM((BPT,), jnp.int32),
        pltpu.VMEM((BPT,), jnp.int32),
        pltpu.VMEM((BPT, 1), jnp.int32),
        pltpu.VMEM((BPT, NEG_PAD), jnp.int32),
        pltpu.VMEM((BPT, 16), jnp.int32),
        pltpu.VMEM((BPT, D), jnp.float32),
        pltpu.VMEM((BPT,), jnp.float32),
        pltpu.VMEM((BPT,), jnp.float32),
        pltpu.VMEM((NEG, D), jnp.float32),
        pltpu.VMEM((17, D), jnp.float32),
        pltpu.VMEM((BPT, NEG_PAD), jnp.float32),
        pltpu.VMEM((BPT, NEG_PAD), jnp.float32),
        pltpu.VMEM((BPT, 16), jnp.float32),
        pltpu.VMEM((NEG_PAD,), jnp.int32),
        pltpu.VMEM((NEG_PAD, 16), jnp.float32),
        pltpu.VMEM((BPT, 16), jnp.float32),
        pltpu.SemaphoreType.DMA,
        pltpu.SemaphoreType.DMA,
    ],
    compiler_params=pltpu.CompilerParams(use_tc_tiling_on_sc=False,
                                         needs_layout_passes=False),
  )


TBL_CHUNK = 5000
TBL_GRID = N_ITEMS // TBL_CHUNK


def _softplus(x):
    return jnp.maximum(x, 0.0) + jnp.log1p(jnp.exp(-jnp.abs(x)))


def _tc_body(ue_ref, ie_ref, nl_ref, nb_ref, sm_ref, nbr_ref, bu_ref,
             bip_ref, out_ref):
    i = pl.program_id(0)

    @pl.when(i == 0)
    def _():
        bu = bu_ref[...]
        neg_w = W3 + W4 * bu * nb_ref[:, :NEG]
        t_neg = jnp.sum(neg_w * _softplus(nl_ref[:, :NEG])) * (NEG_W / NEG)
        pos_w = W1 + W2 * bu * bip_ref[...]
        t_pos = jnp.sum(pos_w * _softplus(-sm_ref[:, 0:1]))
        nbr_f = nbr_ref[:, :K_NBR].astype(jnp.float32)
        t_il = LAMBDA_W * jnp.sum(nbr_f * _softplus(-sm_ref[:, 1:1 + K_NBR]))
        out_ref[0, 0] = t_pos + t_neg + t_il

    norm_part = jnp.sum(ue_ref[...] * ue_ref[...]) + \
        jnp.sum(ie_ref[...] * ie_ref[...])
    out_ref[0, 0] = out_ref[0, 0] + norm_part * (GAMMA * 0.5)


_tc_call = pl.pallas_call(
    _tc_body,
    out_shape=jax.ShapeDtypeStruct((1, 1), jnp.float32),
    grid=(TBL_GRID,),
    in_specs=[
        pl.BlockSpec((TBL_CHUNK, D), lambda i: (i, 0)),
        pl.BlockSpec((TBL_CHUNK, D), lambda i: (i, 0)),
        pl.BlockSpec((B, NEG_PAD), lambda i: (0, 0)),
        pl.BlockSpec((B, NEG_PAD), lambda i: (0, 0)),
        pl.BlockSpec((B, 16), lambda i: (0, 0)),
        pl.BlockSpec((B, 16), lambda i: (0, 0)),
        pl.BlockSpec((B, 1), lambda i: (0, 0)),
        pl.BlockSpec((B, 1), lambda i: (0, 0)),
    ],
    out_specs=pl.BlockSpec(memory_space=pltpu.SMEM),
)


def kernel(user, pos, neg, user_emb, item_emb, beta_uD, beta_iD, il_neighbor):
    neg_pad = jnp.pad(neg, ((0, 0), (0, NEG_PAD - NEG)))
    il_pad = jnp.pad(il_neighbor, ((0, 0), (0, 16 - K_NBR)))
    neg_logit, neg_bi, small, nbr, bu, bip = _get_sc_call()(
        user, pos, pos.reshape(B, 1), neg_pad, user_emb, item_emb,
        beta_uD.reshape(N_USERS // 16, 16), beta_iD.reshape(N_ITEMS // 16, 16),
        il_pad)
    out = _tc_call(user_emb, item_emb, neg_logit, neg_bi, small, nbr,
                   bu.reshape(B, 1), bip.reshape(B, 1))
    return out[0, 0]
